# baseline (speedup 1.0000x reference)
import functools

import jax
import jax.numpy as jnp
from jax import lax
from jax.experimental import pallas as pl
from jax.experimental.pallas import tpu as pltpu
from jax.experimental.pallas import tpu_sc as plsc

B, N, D, C = 4, 1024, 4, 64
F = D * C
K = 20
OUT = 128
BR = 512
NB = N // BR
PTS = B * N

NC, NS, L = 2, 16, 16
NW = NC * NS
PPW = N // NW
G = 8
GK = G * K
NG = PPW // G


def _topk_body(x_ref, w1_ref, out_idx_ref, wmat_ref, dist_ref, *, b):
    r = pl.program_id(0)
    x = x_ref[0]
    w1 = w1_ref[...]
    w1n = w1 * w1 / jnp.sum(w1 * w1)
    wmat_ref[...] = jnp.concatenate([w1n.T] * D, axis=1)

    sq = jnp.sum(x * x, axis=1, keepdims=True)
    row0 = pl.multiple_of(r * BR, BR)
    xb = x_ref[0, pl.ds(row0, BR), :]
    sqb = jnp.sum(xb * xb, axis=1, keepdims=True)
    dist_ref[...] = sqb - 2.0 * lax.dot_general(
        xb, x, (((1,), (1,)), ((), ())), preferred_element_type=jnp.float32
    ) + sq.T

    iota_f = lax.broadcasted_iota(jnp.int32, (BR, N), 1).astype(jnp.float32)
    for k in range(K):
        d = dist_ref[...]
        m = jnp.min(d, axis=1, keepdims=True)
        cand = jnp.where(d <= m, iota_f, float(N))
        idxf = jnp.min(cand, axis=1, keepdims=True)
        msk = cand == idxf
        out_idx_ref[:, k : k + 1] = idxf.astype(jnp.int32) + b * N
        dist_ref[...] = jnp.where(msk, 3.0e38, d)


def _proj_body(wa0_ref, wa1_ref, wa2_ref, wa3_ref, w2_ref, out_ref):
    w2 = w2_ref[...]
    w2n = w2 * w2 / jnp.sum(w2 * w2)
    for i, wa_ref in enumerate((wa0_ref, wa1_ref, wa2_ref, wa3_ref)):
        wa = wa_ref[...]
        for d in range(D):
            out_ref[pl.ds(i * N, N), d * OUT : (d + 1) * OUT] = lax.dot_general(
                wa[:, d * C : (d + 1) * C], w2n, (((1,), (1,)), ((), ())),
                preferred_element_type=jnp.float32)


@functools.partial(
    pl.kernel,
    mesh=plsc.VectorSubcoreMesh(core_axis_name="c", subcore_axis_name="s"),
    out_type=jax.ShapeDtypeStruct((N, F), jnp.float32),
    scratch_types=[
        pltpu.VMEM((PPW * K,), jnp.int32),
        pltpu.VMEM((GK, F), jnp.float32),
        pltpu.VMEM((GK, F), jnp.float32),
        pltpu.VMEM((K, F), jnp.float32),
        pltpu.VMEM((G, F), jnp.float32),
        pltpu.SemaphoreType.DMA,
        pltpu.SemaphoreType.DMA,
    ],
)
def _sc_gather_combine(table_hbm, idx_hbm, wmat_hbm, out_hbm,
                       idx_v, rows0_v, rows1_v, wm_v, acc_v, sem0, sem1):
    wid = lax.axis_index("s") * NC + lax.axis_index("c")
    pltpu.sync_copy(wmat_hbm, wm_v)
    base_pt = wid * PPW
    pltpu.sync_copy(idx_hbm.at[pl.ds(base_pt * K, PPW * K)], idx_v)
    rows = (rows0_v, rows1_v)
    sems = (sem0, sem1)
    H = GK // 2

    def issue(g, slot):
        pltpu.async_copy(table_hbm.at[idx_v.at[pl.ds(g * GK, H)]],
                         rows[slot].at[pl.ds(0, H)], sems[slot])
        pltpu.async_copy(table_hbm.at[idx_v.at[pl.ds(g * GK + H, H)]],
                         rows[slot].at[pl.ds(H, H)], sems[slot])

    def wait(g, slot):
        pltpu.make_async_copy(table_hbm.at[idx_v.at[pl.ds(g * GK, H)]],
                              rows[slot].at[pl.ds(0, H)], sems[slot]).wait()
        pltpu.make_async_copy(table_hbm.at[idx_v.at[pl.ds(g * GK + H, H)]],
                              rows[slot].at[pl.ds(H, H)], sems[slot]).wait()

    def combine_store(g, slot):
        rv = rows[slot]

        def chunk(c, carry2):
            sl = pl.ds(c * L, L)
            ws = [wm_v[k, sl] for k in range(K)]
            for p in range(G):
                acc = rv[p * K, sl] * ws[0]
                for k in range(1, K):
                    acc = acc + rv[p * K + k, sl] * ws[k]
                acc_v[p, sl] = acc
            return carry2

        lax.fori_loop(0, F // L, chunk, 0)
        pltpu.sync_copy(acc_v, out_hbm.at[pl.ds(base_pt + g * G, G)])

    issue(0, 0)
    issue(1, 1)

    def pair(h, carry):
        g = h * 2

        def half(slot):
            gg = g + slot
            wait(gg, slot)
            combine_store(gg, slot)

            @pl.when(gg + 2 < NG)
            def _():
                issue(gg + 2, slot)

        half(0)
        half(1)
        return carry

    lax.fori_loop(0, NG // 2, pair, 0)


def kernel(x, w1, w2):
    x_flat = x.reshape(B, N, F)
    table = x_flat.reshape(PTS, F)
    weighted = []
    for b in range(B):
        idx_b, wmat = pl.pallas_call(
            functools.partial(_topk_body, b=b),
            grid=(NB,),
            in_specs=[
                pl.BlockSpec((1, N, F), lambda r, _b=b: (_b, 0, 0)),
                pl.BlockSpec((C, K), lambda r: (0, 0)),
            ],
            out_specs=[
                pl.BlockSpec((BR, K), lambda r: (r, 0)),
                pl.BlockSpec((K, F), lambda r: (0, 0)),
            ],
            out_shape=[
                jax.ShapeDtypeStruct((N, K), jnp.int32),
                jax.ShapeDtypeStruct((K, F), jnp.float32),
            ],
            scratch_shapes=[pltpu.VMEM((BR, N), jnp.float32)],
        )(x_flat, w1)
        weighted.append(_sc_gather_combine(table, idx_b.reshape(N * K), wmat))

    out = pl.pallas_call(
        _proj_body,
        in_specs=[pl.BlockSpec((N, F), lambda: (0, 0))] * B
        + [pl.BlockSpec((OUT, C), lambda: (0, 0))],
        out_specs=pl.BlockSpec((PTS, D * OUT), lambda: (0, 0)),
        out_shape=jax.ShapeDtypeStruct((PTS, D * OUT), jnp.float32),
    )(*weighted, w2)
    return out.reshape(B, N, D, OUT)


# speedup vs baseline: 1.0470x; 1.0470x over previous
import functools

import jax
import jax.numpy as jnp
from jax import lax
from jax.experimental import pallas as pl
from jax.experimental.pallas import tpu as pltpu
from jax.experimental.pallas import tpu_sc as plsc

B, N, D, C = 4, 1024, 4, 64
F = D * C
K = 20
OUT = 128
BR = 512
NB = N // BR
PTS = B * N

NC, NS, L = 2, 16, 16
NW = NC * NS
PPW = PTS // NW
G = 8
GK = G * K
NG = PPW // G


def _topk_body(x_ref, w1_ref, out_idx_ref, wmat_ref, dist_ref):
    b = pl.program_id(0)
    r = pl.program_id(1)
    x = x_ref[0]
    w1 = w1_ref[...]
    w1n = w1 * w1 / jnp.sum(w1 * w1)
    wmat_ref[...] = jnp.concatenate([w1n.T] * D, axis=1)

    sq = jnp.sum(x * x, axis=1, keepdims=True)
    row0 = pl.multiple_of(r * BR, BR)
    xb = x_ref[0, pl.ds(row0, BR), :]
    sqb = jnp.sum(xb * xb, axis=1, keepdims=True)
    dist_ref[...] = sqb - 2.0 * lax.dot_general(
        xb, x, (((1,), (1,)), ((), ())), preferred_element_type=jnp.float32
    ) + sq.T

    iota_f = lax.broadcasted_iota(jnp.int32, (BR, N), 1).astype(jnp.float32)
    for k in range(K):
        d = dist_ref[...]
        m = jnp.min(d, axis=1, keepdims=True)
        cand = jnp.where(d <= m, iota_f, float(N))
        idxf = jnp.min(cand, axis=1, keepdims=True)
        msk = cand == idxf
        out_idx_ref[0, :, k : k + 1] = idxf.astype(jnp.int32) + b * N
        dist_ref[...] = jnp.where(msk, 3.0e38, d)


def _proj_body(wa_ref, w2_ref, out_ref):
    w2 = w2_ref[...]
    w2n = w2 * w2 / jnp.sum(w2 * w2)
    wa = wa_ref[...]
    for d in range(D):
        out_ref[:, d * OUT : (d + 1) * OUT] = lax.dot_general(
            wa[:, d * C : (d + 1) * C], w2n, (((1,), (1,)), ((), ())),
            preferred_element_type=jnp.float32)


@functools.partial(
    pl.kernel,
    mesh=plsc.VectorSubcoreMesh(core_axis_name="c", subcore_axis_name="s"),
    out_type=jax.ShapeDtypeStruct((PTS, F), jnp.float32),
    scratch_types=[
        pltpu.VMEM((PPW * K,), jnp.int32),
        pltpu.VMEM((GK, F), jnp.float32),
        pltpu.VMEM((GK, F), jnp.float32),
        pltpu.VMEM((K, F), jnp.float32),
        pltpu.VMEM((G, F), jnp.float32),
        pltpu.SemaphoreType.DMA,
        pltpu.SemaphoreType.DMA,
    ],
)
def _sc_gather_combine(table_hbm, idx_hbm, wmat_hbm, out_hbm,
                       idx_v, rows0_v, rows1_v, wm_v, acc_v, sem0, sem1):
    wid = lax.axis_index("s") * NC + lax.axis_index("c")
    pltpu.sync_copy(wmat_hbm, wm_v)
    base_pt = wid * PPW
    pltpu.sync_copy(idx_hbm.at[pl.ds(base_pt * K, PPW * K)], idx_v)
    rows = (rows0_v, rows1_v)
    sems = (sem0, sem1)
    H = GK // 2

    def issue(g, slot):
        pltpu.async_copy(table_hbm.at[idx_v.at[pl.ds(g * GK, H)]],
                         rows[slot].at[pl.ds(0, H)], sems[slot])
        pltpu.async_copy(table_hbm.at[idx_v.at[pl.ds(g * GK + H, H)]],
                         rows[slot].at[pl.ds(H, H)], sems[slot])

    def wait(g, slot):
        pltpu.make_async_copy(table_hbm.at[idx_v.at[pl.ds(g * GK, H)]],
                              rows[slot].at[pl.ds(0, H)], sems[slot]).wait()
        pltpu.make_async_copy(table_hbm.at[idx_v.at[pl.ds(g * GK + H, H)]],
                              rows[slot].at[pl.ds(H, H)], sems[slot]).wait()

    def combine_store(g, slot):
        rv = rows[slot]

        @plsc.parallel_loop(0, F // L, 1, unroll=2)
        def chunk(c):
            sl = pl.ds(c * L, L)
            ws = [wm_v[k, sl] for k in range(K)]
            for p in range(G):
                acc = rv[p * K, sl] * ws[0]
                for k in range(1, K):
                    acc = acc + rv[p * K + k, sl] * ws[k]
                acc_v[p, sl] = acc

        pltpu.sync_copy(acc_v, out_hbm.at[pl.ds(base_pt + g * G, G)])

    issue(0, 0)
    issue(1, 1)

    def pair(h, carry):
        g = h * 2

        def half(slot):
            gg = g + slot
            wait(gg, slot)
            combine_store(gg, slot)

            @pl.when(gg + 2 < NG)
            def _():
                issue(gg + 2, slot)

        half(0)
        half(1)
        return carry

    lax.fori_loop(0, NG // 2, pair, 0)


def kernel(x, w1, w2):
    x_flat = x.reshape(B, N, F)
    idx, wmat = pl.pallas_call(
        _topk_body,
        grid=(B, NB),
        in_specs=[
            pl.BlockSpec((1, N, F), lambda b, r: (b, 0, 0)),
            pl.BlockSpec((C, K), lambda b, r: (0, 0)),
        ],
        out_specs=[
            pl.BlockSpec((1, BR, K), lambda b, r: (b, r, 0)),
            pl.BlockSpec((K, F), lambda b, r: (0, 0)),
        ],
        out_shape=[
            jax.ShapeDtypeStruct((B, N, K), jnp.int32),
            jax.ShapeDtypeStruct((K, F), jnp.float32),
        ],
        scratch_shapes=[pltpu.VMEM((BR, N), jnp.float32)],
    )(x_flat, w1)

    weighted = _sc_gather_combine(
        x_flat.reshape(PTS, F), idx.reshape(PTS * K), wmat)

    out = pl.pallas_call(
        _proj_body,
        in_specs=[
            pl.BlockSpec((PTS, F), lambda: (0, 0)),
            pl.BlockSpec((OUT, C), lambda: (0, 0)),
        ],
        out_specs=pl.BlockSpec((PTS, D * OUT), lambda: (0, 0)),
        out_shape=jax.ShapeDtypeStruct((PTS, D * OUT), jnp.float32),
    )(weighted, w2)
    return out.reshape(B, N, D, OUT)


# speedup vs baseline: 1.0480x; 1.0009x over previous
import functools

import jax
import jax.numpy as jnp
from jax import lax
from jax.experimental import pallas as pl
from jax.experimental.pallas import tpu as pltpu
from jax.experimental.pallas import tpu_sc as plsc

B, N, D, C = 4, 1024, 4, 64
F = D * C
K = 20
OUT = 128
BR = 512
NB = N // BR
PTS = B * N

NC, NS, L = 2, 16, 16
NW = NC * NS
PPW = PTS // NW
G = 8
GK = G * K
NG = PPW // G


def _topk_body(x_ref, w1_ref, out_idx_ref, wmat_ref, dist_ref):
    b = pl.program_id(0)
    r = pl.program_id(1)
    x = x_ref[0]
    w1 = w1_ref[...]
    w1n = w1 * w1 / jnp.sum(w1 * w1)
    wmat_ref[...] = jnp.concatenate([w1n.T] * D, axis=1)

    sq = jnp.sum(x * x, axis=1, keepdims=True)
    row0 = pl.multiple_of(r * BR, BR)
    xb = x_ref[0, pl.ds(row0, BR), :]
    sqb = jnp.sum(xb * xb, axis=1, keepdims=True)
    dist_ref[...] = sqb - 2.0 * lax.dot_general(
        xb, x, (((1,), (1,)), ((), ())), preferred_element_type=jnp.float32
    ) + sq.T

    iota_f = lax.broadcasted_iota(jnp.int32, (BR, N), 1).astype(jnp.float32)
    for k in range(K):
        d = dist_ref[...]
        m = jnp.min(d, axis=1, keepdims=True)
        cand = jnp.where(d <= m, iota_f, float(N))
        idxf = jnp.min(cand, axis=1, keepdims=True)
        msk = cand == idxf
        out_idx_ref[0, :, k : k + 1] = idxf.astype(jnp.int32) + b * N
        dist_ref[...] = jnp.where(msk, 3.0e38, d)


def _proj_body(wa_ref, w2_ref, out_ref):
    w2 = w2_ref[...]
    w2n = w2 * w2 / jnp.sum(w2 * w2)
    wa = wa_ref[...]
    for d in range(D):
        out_ref[:, d * OUT : (d + 1) * OUT] = lax.dot_general(
            wa[:, d * C : (d + 1) * C], w2n, (((1,), (1,)), ((), ())),
            preferred_element_type=jnp.float32)


@functools.partial(
    pl.kernel,
    mesh=plsc.VectorSubcoreMesh(core_axis_name="c", subcore_axis_name="s"),
    out_type=jax.ShapeDtypeStruct((PTS, F), jnp.float32),
    scratch_types=[
        pltpu.VMEM((PPW * K,), jnp.int32),
        pltpu.VMEM((GK, F), jnp.float32),
        pltpu.VMEM((GK, F), jnp.float32),
        pltpu.VMEM((K, F), jnp.float32),
        pltpu.VMEM((G, F), jnp.float32),
        pltpu.VMEM((G, F), jnp.float32),
        pltpu.SemaphoreType.DMA,
        pltpu.SemaphoreType.DMA,
        pltpu.SemaphoreType.DMA,
        pltpu.SemaphoreType.DMA,
    ],
)
def _sc_gather_combine(table_hbm, idx_hbm, wmat_hbm, out_hbm,
                       idx_v, rows0_v, rows1_v, wm_v, acc0_v, acc1_v,
                       sem0, sem1, osem0, osem1):
    wid = lax.axis_index("s") * NC + lax.axis_index("c")
    pltpu.sync_copy(wmat_hbm, wm_v)
    base_pt = wid * PPW
    pltpu.sync_copy(idx_hbm.at[pl.ds(base_pt * K, PPW * K)], idx_v)
    rows = (rows0_v, rows1_v)
    sems = (sem0, sem1)
    accs = (acc0_v, acc1_v)
    osems = (osem0, osem1)
    H = GK // 2

    def issue(g, slot):
        pltpu.async_copy(table_hbm.at[idx_v.at[pl.ds(g * GK, H)]],
                         rows[slot].at[pl.ds(0, H)], sems[slot])
        pltpu.async_copy(table_hbm.at[idx_v.at[pl.ds(g * GK + H, H)]],
                         rows[slot].at[pl.ds(H, H)], sems[slot])

    def wait(g, slot):
        pltpu.make_async_copy(table_hbm.at[idx_v.at[pl.ds(g * GK, H)]],
                              rows[slot].at[pl.ds(0, H)], sems[slot]).wait()
        pltpu.make_async_copy(table_hbm.at[idx_v.at[pl.ds(g * GK + H, H)]],
                              rows[slot].at[pl.ds(H, H)], sems[slot]).wait()

    def combine_store(g, slot):
        rv = rows[slot]
        av = accs[slot]

        @pl.when(g >= 2)
        def _():
            pltpu.make_async_copy(
                av, out_hbm.at[pl.ds(base_pt + (g - 2) * G, G)], osems[slot]
            ).wait()

        @plsc.parallel_loop(0, F // L, 1, unroll=2)
        def chunk(c):
            sl = pl.ds(c * L, L)
            ws = [wm_v[k, sl] for k in range(K)]
            for p in range(G):
                acc = rv[p * K, sl] * ws[0]
                for k in range(1, K):
                    acc = acc + rv[p * K + k, sl] * ws[k]
                av[p, sl] = acc

        pltpu.async_copy(av, out_hbm.at[pl.ds(base_pt + g * G, G)], osems[slot])

    issue(0, 0)
    issue(1, 1)

    def pair(h, carry):
        g = h * 2

        def half(slot):
            gg = g + slot
            wait(gg, slot)
            combine_store(gg, slot)

            @pl.when(gg + 2 < NG)
            def _():
                issue(gg + 2, slot)

        half(0)
        half(1)
        return carry

    lax.fori_loop(0, NG // 2, pair, 0)

    pltpu.make_async_copy(
        accs[0], out_hbm.at[pl.ds(base_pt + (NG - 2) * G, G)], osems[0]).wait()
    pltpu.make_async_copy(
        accs[1], out_hbm.at[pl.ds(base_pt + (NG - 1) * G, G)], osems[1]).wait()


def kernel(x, w1, w2):
    x_flat = x.reshape(B, N, F)
    idx, wmat = pl.pallas_call(
        _topk_body,
        grid=(B, NB),
        in_specs=[
            pl.BlockSpec((1, N, F), lambda b, r: (b, 0, 0)),
            pl.BlockSpec((C, K), lambda b, r: (0, 0)),
        ],
        out_specs=[
            pl.BlockSpec((1, BR, K), lambda b, r: (b, r, 0)),
            pl.BlockSpec((K, F), lambda b, r: (0, 0)),
        ],
        out_shape=[
            jax.ShapeDtypeStruct((B, N, K), jnp.int32),
            jax.ShapeDtypeStruct((K, F), jnp.float32),
        ],
        scratch_shapes=[pltpu.VMEM((BR, N), jnp.float32)],
    )(x_flat, w1)

    weighted = _sc_gather_combine(
        x_flat.reshape(PTS, F), idx.reshape(PTS * K), wmat)

    out = pl.pallas_call(
        _proj_body,
        in_specs=[
            pl.BlockSpec((PTS, F), lambda: (0, 0)),
            pl.BlockSpec((OUT, C), lambda: (0, 0)),
        ],
        out_specs=pl.BlockSpec((PTS, D * OUT), lambda: (0, 0)),
        out_shape=jax.ShapeDtypeStruct((PTS, D * OUT), jnp.float32),
    )(weighted, w2)
    return out.reshape(B, N, D, OUT)
